# Initial kernel scaffold; baseline (speedup 1.0000x reference)
import jax
import jax.numpy as jnp
from jax.experimental import pallas as pl


def kernel(target_ad, ubs_feature, profile_feature, context_feature, item_emb, profile_emb, W, b):
    raise NotImplementedError("write your pallas kernel here")


# speedup vs baseline: 7.9677x; 7.9677x over previous
import functools

import jax
import jax.numpy as jnp
from jax import lax
from jax.experimental import pallas as pl
from jax.experimental.pallas import tpu as pltpu
from jax.experimental.pallas import tpu_sc as plsc

B = 4096
L = 50
F = 13
V = 100000
D = 16
C = 16

NC = 2
NS = 16
NW = NC * NS
BPW = B // NW
G = 128
NUBS = 2 * F
HALF = B // 2

VROWS = V // 8
SPLIT = 4
RB = VROWS // SPLIT


def _tables_body(item_ref, prof_ref, wt_ref, wu_ref, wp_ref,
                 ot_ref, ou_ref, op_ref):
    a = item_ref[0]
    p = prof_ref[0]
    ot_ref[0] = jnp.dot(a, wt_ref[0], preferred_element_type=jnp.float32)
    ou_ref[0] = jnp.dot(a, wu_ref[0], preferred_element_type=jnp.float32)
    op_ref[0] = jnp.dot(p, wp_ref[0], preferred_element_type=jnp.float32)


def _build_tables(item2, prof2, wm_t, wm_u, wm_p):
    grid = (F * SPLIT,)
    emb_spec = pl.BlockSpec((1, RB, 128), lambda i: (i, 0, 0))
    w_spec = pl.BlockSpec((1, 128, 8), lambda i: (i, 0, 0))
    out_spec = pl.BlockSpec((1, RB, 8), lambda i: (i, 0, 0))
    out_shape = jax.ShapeDtypeStruct((F * SPLIT, RB, 8), jnp.float32)
    return pl.pallas_call(
        _tables_body,
        grid=grid,
        in_specs=[emb_spec, emb_spec, w_spec, w_spec, w_spec],
        out_specs=[out_spec, out_spec, out_spec],
        out_shape=[out_shape, out_shape, out_shape],
    )(item2, prof2, wm_t, wm_u, wm_p)


def _sc_body(ubs_t, tgt_f, prof_f, tub, ttgt, tprof, out,
             table_v, gidx_v, idx_v, val_v, acc_v, sem):
    wid = lax.axis_index("s") * NC + lax.axis_index("c")
    b0 = wid * BPW

    def zero_one(i, _):
        acc_v[pl.ds(i * 16, 16)] = jnp.zeros((16,), jnp.float32)
        return 0

    lax.fori_loop(0, B // 16, zero_one, 0)

    def flat_pass(idx_src, tab_flat):
        def fbody(f, _):
            pltpu.sync_copy(idx_src.at[pl.ds(f * B + b0, BPW)], idx_v)

            def add_one(k, _):
                s = pl.ds(k * 16, 16)
                idx_v[s] = idx_v[s] + f * V
                return 0

            lax.fori_loop(0, BPW // 16, add_one, 0)
            pltpu.async_copy(tab_flat.at[idx_v], val_v, sem).wait()

            def acc_one(k, _):
                d = pl.ds(b0 + k * 16, 16)
                acc_v[d] = acc_v[d] + val_v[pl.ds(k * 16, 16)]
                return 0

            lax.fori_loop(0, BPW // 16, acc_one, 0)
            return 0

        lax.fori_loop(0, F, fbody, 0)

    flat_pass(tgt_f, ttgt)
    flat_pass(prof_f, tprof)

    @pl.when(wid < NUBS)
    def _():
        f = wid // 2
        base = (wid % 2) * HALF
        pltpu.sync_copy(tub.at[pl.ds(f * V, V)], table_v)

        def gbody(g, _):
            bb = base + g * G
            pltpu.sync_copy(ubs_t.at[f, :, pl.ds(bb, G)], gidx_v)

            def kbody(k, _):
                s16 = jnp.zeros((16,), jnp.float32)
                for l in range(L):
                    s16 = s16 + plsc.load_gather(
                        table_v, [gidx_v[l, pl.ds(k * 16, 16)]])
                d = pl.ds(bb + k * 16, 16)
                acc_v[d] = acc_v[d] + s16
                return 0

            lax.fori_loop(0, G // 16, kbody, 0)
            return 0

        lax.fori_loop(0, HALF // G, gbody, 0)

    pltpu.sync_copy(acc_v, out.at[pl.ds(wid * B, B)])


@functools.cache
def _sc_gather_fn():
    return functools.partial(
        pl.kernel,
        out_type=jax.ShapeDtypeStruct((NW * B,), jnp.float32),
        mesh=plsc.VectorSubcoreMesh(core_axis_name="c", subcore_axis_name="s",
                                    num_cores=NC, num_subcores=NS),
        scratch_types=[
            pltpu.VMEM((V,), jnp.float32),
            pltpu.VMEM((L, G), jnp.int32),
            pltpu.VMEM((BPW,), jnp.int32),
            pltpu.VMEM((BPW,), jnp.float32),
            pltpu.VMEM((B,), jnp.float32),
            pltpu.SemaphoreType.DMA,
        ],
        compiler_params=pltpu.CompilerParams(needs_layout_passes=False),
    )(_sc_body)


def _head_body(p_ref, ctx_ref, wc_ref, b_ref, o_ref):
    s = jnp.sum(p_ref[...], axis=0)
    c = jnp.dot(ctx_ref[...], wc_ref[...], preferred_element_type=jnp.float32)
    logit = s[:, None] + c + b_ref[0, 0]
    o_ref[...] = jax.nn.sigmoid(logit)


def _head(partials, context, wc, bias):
    return pl.pallas_call(
        _head_body,
        out_shape=jax.ShapeDtypeStruct((B, 1), jnp.float32),
    )(partials, context, wc, bias)


def kernel(target_ad, ubs_feature, profile_feature, context_feature,
           item_emb, profile_emb, W, b):
    wt = W[:F * D, 0].reshape(F, D)
    wu = W[F * D:2 * F * D, 0].reshape(F, D) / L
    wp = W[2 * F * D:3 * F * D, 0].reshape(F, D)
    wc = W[3 * F * D:, :]
    eye8 = jnp.eye(8, dtype=jnp.float32)

    def expand(w):
        wm = (eye8[None, :, None, :] * w[:, None, :, None]).reshape(F, 128, 8)
        return jnp.repeat(wm, SPLIT, axis=0)

    item2 = item_emb.reshape(F * SPLIT, RB, 128)
    prof2 = profile_emb.reshape(F * SPLIT, RB, 128)
    t_tgt8, t_ubs8, t_prof8 = _build_tables(
        item2, prof2, expand(wt), expand(wu), expand(wp))

    tub = t_ubs8.reshape(F * V)
    ttgt = t_tgt8.reshape(F * V)
    tprof = t_prof8.reshape(F * V)

    ubs_t = jnp.transpose(ubs_feature, (2, 1, 0))
    tgt_f = target_ad.T.reshape(F * B)
    prof_f = profile_feature.T.reshape(F * B)

    partials = _sc_gather_fn()(ubs_t, tgt_f, prof_f, tub, ttgt, tprof)

    return _head(partials.reshape(NW, B), context_feature, wc, b.reshape(1, 1))
